# baseline (speedup 1.0000x reference)
import functools

import jax
import jax.numpy as jnp
from jax import lax
from jax.experimental import pallas as pl
from jax.experimental.pallas import tpu as pltpu
from jax.experimental.pallas import tpu_sc as plsc

K = 1024
D = 64
DP = 128
B0 = 16
N = B0 * 1024
BN = 1024
HB = B0 // 2
NB = HB
NH = N // 2
NC = 2
NS = 16
NW = NC * NS
BPW = NH // NW
WPB = BN // BPW
GCH = 128
NCH = BPW // GCH
COMMIT = 0.25


def _tc_dist_body(x_ref, cb_ref, idx_ref, cbp_ref, lsum_ref, acc_ref):
    i = pl.program_id(0)
    xT = x_ref[0]
    cbT = cb_ref[...]
    s = lax.dot_general(xT, cbT, (((0,), (0,)), ((), ())),
                        preferred_element_type=jnp.float32)
    x2 = jnp.sum(xT * xT, axis=0)
    c2 = jnp.sum(cbT * cbT, axis=0)
    d = x2[:, None] - 2.0 * s + c2[None, :]
    minv = jnp.min(d, axis=1)
    iota_f = lax.broadcasted_iota(jnp.int32, (BN, K), 1).astype(jnp.float32)
    cand = jnp.where(d == minv[:, None], iota_f, jnp.float32(K))
    idx = jnp.min(cand, axis=1).astype(jnp.int32)
    idx_ref[...] = idx.reshape(WPB, NCH, GCH)
    bsum = jnp.sum(minv)

    @pl.when(i == 0)
    def _():
        acc_ref[0, 0] = 0.0
        cbp_ref[...] = jnp.concatenate(
            [cbT.T, jnp.zeros((K, DP - D), jnp.float32)], axis=1)

    acc_ref[0, 0] += bsum

    @pl.when(i == NB - 1)
    def _():
        lsum_ref[0, 0] = acc_ref[0, 0]


_tc_dist = pl.pallas_call(
    _tc_dist_body,
    grid=(NB,),
    in_specs=[
        pl.BlockSpec((1, D, BN), lambda i: (i, 0, 0)),
        pl.BlockSpec((D, K), lambda i: (0, 0)),
    ],
    out_specs=[
        pl.BlockSpec((WPB, NCH, GCH), lambda i: (i, 0, 0)),
        pl.BlockSpec((K, DP), lambda i: (0, 0)),
        pl.BlockSpec(memory_space=pltpu.SMEM),
    ],
    out_shape=[
        jax.ShapeDtypeStruct((NW, NCH, GCH), jnp.int32),
        jax.ShapeDtypeStruct((K, DP), jnp.float32),
        jax.ShapeDtypeStruct((1, 1), jnp.float32),
    ],
    scratch_shapes=[pltpu.SMEM((1, 1), jnp.float32)],
)


def _sc_body(idx_hbm, cb_hbm, out_hbm, counts_hbm,
             idx_v, rows_v, ones_v, zer_v, shared_cnt, gsem, osem, hsem):
    c = lax.axis_index("c")
    s = lax.axis_index("s")
    wid = s * NC + c
    pltpu.sync_copy(idx_hbm.at[wid], idx_v)
    gathers = [
        pltpu.async_copy(cb_hbm.at[idx_v.at[j]],
                         rows_v.at[pl.ds(j * GCH, GCH)], gsem)
        for j in range(NCH)
    ]
    for t in range(K // 16):
        zer_v[pl.ds(t * 16, 16)] = jnp.zeros((16,), jnp.float32)
    for t in range(GCH // 16):
        ones_v[pl.ds(t * 16, 16)] = jnp.ones((16,), jnp.float32)

    @pl.when(s == 0)
    def _():
        pltpu.sync_copy(zer_v, shared_cnt)

    plsc.subcore_barrier()
    hists = [
        pltpu.async_copy(ones_v, shared_cnt.at[idx_v.at[j]], add=True,
                         sem=hsem)
        for j in range(NCH)
    ]
    outs = []
    for j in range(NCH):
        gathers[j].wait()
        outs.append(pltpu.async_copy(
            rows_v.at[pl.ds(j * GCH, GCH)],
            out_hbm.at[pl.ds(wid * BPW + j * GCH, GCH)], osem))
    for h in hists:
        h.wait()
    for o in outs:
        o.wait()
    plsc.subcore_barrier()

    @pl.when(s == 0)
    def _():
        pltpu.sync_copy(shared_cnt, counts_hbm.at[c])


@functools.cache
def _sc_gather_hist():
    mesh = plsc.VectorSubcoreMesh(
        core_axis_name="c", subcore_axis_name="s",
        num_cores=NC, num_subcores=NS)
    return pl.kernel(
        _sc_body,
        out_type=(
            jax.ShapeDtypeStruct((NH, DP), jnp.float32),
            jax.ShapeDtypeStruct((NC, K), jnp.float32),
        ),
        mesh=mesh,
        scratch_types=[
            pltpu.VMEM((NCH, GCH), jnp.int32),
            pltpu.VMEM((BPW, DP), jnp.float32),
            pltpu.VMEM((GCH,), jnp.float32),
            pltpu.VMEM((K,), jnp.float32),
            pltpu.VMEM_SHARED((K,), jnp.float32),
            pltpu.SemaphoreType.DMA,
            pltpu.SemaphoreType.DMA,
            pltpu.SemaphoreType.DMA,
        ],
    )


SPF = 2
NB3 = B0 // SPF
HB3 = NB3 // 2


def _tc_fin_body(rowsa_ref, rowsb_ref, cnta_ref, cntb_ref,
                 suma_ref, sumb_ref, out_ref, loss_ref, perp_ref):
    i = pl.program_id(0)
    for t in range(SPF):

        @pl.when(i < HB3)
        def _():
            out_ref[t] = rowsa_ref[pl.ds(t * 1024, 1024), :D].T

        @pl.when(i >= HB3)
        def _():
            out_ref[t] = rowsb_ref[pl.ds(t * 1024, 1024), :D].T

    @pl.when(i == NB3 - 1)
    def _():
        loss_ref[0, 0] = ((1.0 + COMMIT) / (N * D)) * (
            suma_ref[0, 0] + sumb_ref[0, 0])
        cnt = cnta_ref[...] + cntb_ref[...]
        p = jnp.sum(cnt, axis=0) * (1.0 / N)
        ent = jnp.sum(p * -jnp.log(p + 1e-10))
        perp_ref[0, 0] = jnp.exp(ent)


_tc_fin = pl.pallas_call(
    _tc_fin_body,
    grid=(NB3,),
    in_specs=[
        pl.BlockSpec((SPF * 1024, DP), lambda i: (jnp.minimum(i, HB3 - 1), 0)),
        pl.BlockSpec((SPF * 1024, DP),
                     lambda i: (jnp.maximum(i - HB3, 0), 0)),
        pl.BlockSpec((NC, K), lambda i: (0, 0)),
        pl.BlockSpec((NC, K), lambda i: (0, 0)),
        pl.BlockSpec(memory_space=pltpu.SMEM),
        pl.BlockSpec(memory_space=pltpu.SMEM),
    ],
    out_specs=[
        pl.BlockSpec((SPF, D, 1024), lambda i: (i, 0, 0)),
        pl.BlockSpec(memory_space=pltpu.SMEM),
        pl.BlockSpec(memory_space=pltpu.SMEM),
    ],
    out_shape=[
        jax.ShapeDtypeStruct((B0, D, 1024), jnp.float32),
        jax.ShapeDtypeStruct((1, 1), jnp.float32),
        jax.ShapeDtypeStruct((1, 1), jnp.float32),
    ],
)


def kernel(inputs, codebook):
    t_in = jnp.transpose(inputs, (0, 2, 1))
    cbT = codebook.T
    sc = _sc_gather_hist()
    idx_a, cb_pad, sum_a = _tc_dist(t_in[:HB], cbT)
    rows_a, cnt_a = sc(idx_a, cb_pad)
    idx_b, _, sum_b = _tc_dist(t_in[HB:], cbT)
    rows_b, cnt_b = sc(idx_b, cb_pad)
    qT, loss_arr, perp = _tc_fin(rows_a, rows_b, cnt_a, cnt_b, sum_a, sum_b)
    quant = jnp.transpose(qT, (0, 2, 1))
    return quant, loss_arr[0, 0], perp[0, 0]


# speedup vs baseline: 1.0414x; 1.0414x over previous
import functools

import jax
import jax.numpy as jnp
from jax import lax
from jax.experimental import pallas as pl
from jax.experimental.pallas import tpu as pltpu
from jax.experimental.pallas import tpu_sc as plsc

K = 1024
D = 64
DP = 128
B0 = 16
N = B0 * 1024
BN = 1024
HB = B0 // 2
NB = HB
NH = N // 2
NC = 2
NS = 16
NW = NC * NS
BPW = NH // NW
WPB = BN // BPW
GCH = 128
NCH = BPW // GCH
COMMIT = 0.25


def _tc_dist_body(x_ref, cb_ref, idx_ref, cbp_ref, lsum_ref, acc_ref):
    i = pl.program_id(0)
    xT = x_ref[0]
    cbT = cb_ref[...]
    s = lax.dot_general(xT, cbT, (((0,), (0,)), ((), ())),
                        preferred_element_type=jnp.float32)
    x2 = jnp.sum(xT * xT, axis=0)
    c2 = jnp.sum(cbT * cbT, axis=0)
    d = x2[:, None] - 2.0 * s + c2[None, :]
    minv = jnp.min(d, axis=1)
    iota_f = lax.broadcasted_iota(jnp.int32, (BN, K), 1).astype(jnp.float32)
    cand = jnp.where(d == minv[:, None], iota_f, jnp.float32(K))
    idx = jnp.min(cand, axis=1).astype(jnp.int32)
    idx_ref[...] = idx.reshape(WPB, NCH, GCH)
    bsum = jnp.sum(minv)

    @pl.when(i == 0)
    def _():
        acc_ref[0, 0] = 0.0
        cbp_ref[...] = jnp.concatenate(
            [cbT.T, jnp.zeros((K, DP - D), jnp.float32)], axis=1)

    acc_ref[0, 0] += bsum

    @pl.when(i == NB - 1)
    def _():
        lsum_ref[0, 0] = acc_ref[0, 0]


_tc_dist = pl.pallas_call(
    _tc_dist_body,
    grid=(NB,),
    in_specs=[
        pl.BlockSpec((1, D, BN), lambda i: (i, 0, 0)),
        pl.BlockSpec((D, K), lambda i: (0, 0)),
    ],
    out_specs=[
        pl.BlockSpec((WPB, NCH, GCH), lambda i: (i, 0, 0)),
        pl.BlockSpec((K, DP), lambda i: (0, 0)),
        pl.BlockSpec(memory_space=pltpu.SMEM),
    ],
    out_shape=[
        jax.ShapeDtypeStruct((NW, NCH, GCH), jnp.int32),
        jax.ShapeDtypeStruct((K, DP), jnp.float32),
        jax.ShapeDtypeStruct((1, 1), jnp.float32),
    ],
    scratch_shapes=[pltpu.SMEM((1, 1), jnp.float32)],
)


def _sc_body(idx_hbm, cb_hbm, out_hbm, counts_hbm,
             idx_v, rows_v, ones_v, zer_v, shared_cnt, gsem, osem, hsem):
    c = lax.axis_index("c")
    s = lax.axis_index("s")
    wid = s * NC + c
    pltpu.sync_copy(idx_hbm.at[wid], idx_v)
    gathers = [
        pltpu.async_copy(cb_hbm.at[idx_v.at[j]],
                         rows_v.at[pl.ds(j * GCH, GCH)], gsem)
        for j in range(NCH)
    ]
    for t in range(K // 16):
        zer_v[pl.ds(t * 16, 16)] = jnp.zeros((16,), jnp.float32)
    for t in range(GCH // 16):
        ones_v[pl.ds(t * 16, 16)] = jnp.ones((16,), jnp.float32)

    @pl.when(s == 0)
    def _():
        pltpu.sync_copy(zer_v, shared_cnt)

    plsc.subcore_barrier()
    hists = [
        pltpu.async_copy(ones_v, shared_cnt.at[idx_v.at[j]], add=True,
                         sem=hsem)
        for j in range(NCH)
    ]
    outs = []
    for j in range(NCH):
        gathers[j].wait()
        outs.append(pltpu.async_copy(
            rows_v.at[pl.ds(j * GCH, GCH)],
            out_hbm.at[pl.ds(wid * BPW + j * GCH, GCH)], osem))
    for h in hists:
        h.wait()
    for o in outs:
        o.wait()
    plsc.subcore_barrier()

    @pl.when(s == 0)
    def _():
        pltpu.sync_copy(shared_cnt, counts_hbm.at[c])


@functools.cache
def _sc_gather_hist():
    mesh = plsc.VectorSubcoreMesh(
        core_axis_name="c", subcore_axis_name="s",
        num_cores=NC, num_subcores=NS)
    return pl.kernel(
        _sc_body,
        out_type=(
            jax.ShapeDtypeStruct((NH, DP), jnp.float32),
            jax.ShapeDtypeStruct((NC, K), jnp.float32),
        ),
        mesh=mesh,
        scratch_types=[
            pltpu.VMEM((NCH, GCH), jnp.int32),
            pltpu.VMEM((BPW, DP), jnp.float32),
            pltpu.VMEM((GCH,), jnp.float32),
            pltpu.VMEM((K,), jnp.float32),
            pltpu.VMEM_SHARED((K,), jnp.float32),
            pltpu.SemaphoreType.DMA,
            pltpu.SemaphoreType.DMA,
            pltpu.SemaphoreType.DMA,
        ],
    )


SPF = 2
NB3 = B0 // SPF
HB3 = NB3 // 2


def _tc_fina_body(rows_ref, out_ref):
    for t in range(SPF):
        out_ref[t] = rows_ref[pl.ds(t * 1024, 1024), :D].T


_tc_fina = pl.pallas_call(
    _tc_fina_body,
    grid=(HB3,),
    in_specs=[pl.BlockSpec((SPF * 1024, DP), lambda i: (i, 0))],
    out_specs=pl.BlockSpec((SPF, D, 1024), lambda i: (i, 0, 0)),
    out_shape=jax.ShapeDtypeStruct((B0, D, 1024), jnp.float32),
)


def _tc_finb_body(rows_ref, cnta_ref, cntb_ref, suma_ref, sumb_ref,
                  qpart_ref, out_ref, loss_ref, perp_ref):
    i = pl.program_id(0)
    del qpart_ref
    for t in range(SPF):
        out_ref[t] = rows_ref[pl.ds(t * 1024, 1024), :D].T

    @pl.when(i == HB3 - 1)
    def _():
        loss_ref[0, 0] = ((1.0 + COMMIT) / (N * D)) * (
            suma_ref[0, 0] + sumb_ref[0, 0])
        cnt = cnta_ref[...] + cntb_ref[...]
        p = jnp.sum(cnt, axis=0) * (1.0 / N)
        ent = jnp.sum(p * -jnp.log(p + 1e-10))
        perp_ref[0, 0] = jnp.exp(ent)


_tc_finb = pl.pallas_call(
    _tc_finb_body,
    grid=(HB3,),
    in_specs=[
        pl.BlockSpec((SPF * 1024, DP), lambda i: (i, 0)),
        pl.BlockSpec((NC, K), lambda i: (0, 0)),
        pl.BlockSpec((NC, K), lambda i: (0, 0)),
        pl.BlockSpec(memory_space=pltpu.SMEM),
        pl.BlockSpec(memory_space=pltpu.SMEM),
        pl.BlockSpec(memory_space=pl.ANY),
    ],
    out_specs=[
        pl.BlockSpec((SPF, D, 1024), lambda i: (i + HB3, 0, 0)),
        pl.BlockSpec(memory_space=pltpu.SMEM),
        pl.BlockSpec(memory_space=pltpu.SMEM),
    ],
    out_shape=[
        jax.ShapeDtypeStruct((B0, D, 1024), jnp.float32),
        jax.ShapeDtypeStruct((1, 1), jnp.float32),
        jax.ShapeDtypeStruct((1, 1), jnp.float32),
    ],
    input_output_aliases={5: 0},
)


def kernel(inputs, codebook):
    t_in = jnp.transpose(inputs, (0, 2, 1))
    cbT = codebook.T
    sc = _sc_gather_hist()
    idx_a, cb_pad, sum_a = _tc_dist(t_in[:HB], cbT)
    rows_a, cnt_a = sc(idx_a, cb_pad)
    idx_b, _, sum_b = _tc_dist(t_in[HB:], cbT)
    rows_b, cnt_b = sc(idx_b, cb_pad)
    q_half = _tc_fina(rows_a)
    qT, loss_arr, perp = _tc_finb(rows_b, cnt_a, cnt_b, sum_a, sum_b, q_half)
    quant = jnp.transpose(qT, (0, 2, 1))
    return quant, loss_arr[0, 0], perp[0, 0]
